# baseline (speedup 1.0000x reference)
import functools

import jax
import jax.numpy as jnp
from jax import lax
from jax.experimental import pallas as pl
from jax.experimental.pallas import tpu as pltpu
from jax.experimental.pallas import tpu_sc as plsc

_D = 128
_B = 1024
_P = 50
_R = _B * _P * 2
_NC, _NS = 2, 16
_NW = _NC * _NS
_C = 128
_PER_W = _R // _NW
_NCHUNK = _PER_W // _C

_mesh = plsc.VectorSubcoreMesh(core_axis_name="c", subcore_axis_name="s")


@functools.partial(
    pl.kernel,
    out_type=jax.ShapeDtypeStruct((_R, _D), jnp.float32),
    mesh=_mesh,
    scratch_types=[
        pltpu.VMEM((_PER_W,), jnp.int32),
        pltpu.VMEM((4, _C, _D), jnp.float32),
        pltpu.SemaphoreType.DMA,
        pltpu.SemaphoreType.DMA,
    ],
)
def _gather(table_hbm, idx_hbm, out_hbm, idx_v, rows_v, gsem, osem):
    wid = lax.axis_index("s") * _NC + lax.axis_index("c")
    base = wid * _PER_W
    pltpu.sync_copy(idx_hbm.at[pl.ds(base, _PER_W)], idx_v)
    pltpu.async_copy(table_hbm.at[idx_v.at[pl.ds(0, _C)]], rows_v.at[0], gsem)
    pltpu.async_copy(table_hbm.at[idx_v.at[pl.ds(_C, _C)]], rows_v.at[1], gsem)

    @pl.loop(0, _NCHUNK)
    def _chunk(j):
        buf = lax.rem(j, 4)
        pltpu.make_async_copy(
            table_hbm.at[idx_v.at[pl.ds(j * _C, _C)]], rows_v.at[buf], gsem
        ).wait()
        pltpu.async_copy(
            rows_v.at[buf], out_hbm.at[pl.ds(base + j * _C, _C)], osem
        )

        @pl.when(j + 2 < _NCHUNK)
        def _start_next():
            nbuf = lax.rem(j + 2, 4)

            @pl.when(j >= 2)
            def _reclaim():
                pltpu.make_async_copy(
                    rows_v.at[nbuf],
                    out_hbm.at[pl.ds(base + (j - 2) * _C, _C)],
                    osem,
                ).wait()

            pltpu.async_copy(
                table_hbm.at[idx_v.at[pl.ds((j + 2) * _C, _C)]],
                rows_v.at[nbuf],
                gsem,
            )

    for _ in range(4):
        pltpu.make_async_copy(
            rows_v.at[0], out_hbm.at[pl.ds(base, _C)], osem
        ).wait()


def kernel(table, idx):
    idx_flat = idx.reshape(_R)
    out = _gather(table, idx_flat)
    return out.reshape(_B, _P, 1, 2, _D)


# speedup vs baseline: 1.0011x; 1.0011x over previous
import functools

import jax
import jax.numpy as jnp
from jax import lax
from jax.experimental import pallas as pl
from jax.experimental.pallas import tpu as pltpu
from jax.experimental.pallas import tpu_sc as plsc

_D = 128
_B = 1024
_P = 50
_R = _B * _P * 2
_NC, _NS = 2, 16
_NW = _NC * _NS
_C = 128
_NROW = _R // _C
_NCHUNK = _NROW // _NW
_PER_W = _NCHUNK * _C
_STAGE = 32

_mesh = plsc.VectorSubcoreMesh(core_axis_name="c", subcore_axis_name="s")


@functools.partial(
    pl.kernel,
    out_type=jax.ShapeDtypeStruct((_R, _D), jnp.float32),
    mesh=_mesh,
    scratch_types=[
        pltpu.VMEM((_STAGE, _C), jnp.int32),
        pltpu.VMEM((4, _C, _D), jnp.float32),
        pltpu.SemaphoreType.DMA,
        pltpu.SemaphoreType.DMA,
    ],
)
def _gather(table_hbm, idx_hbm, out_hbm, idx_v, rows_v, gsem, osem):
    wid = lax.axis_index("s") * _NC + lax.axis_index("c")
    row0 = wid * _NCHUNK
    aligned = (row0 // 8) * 8
    off = row0 - aligned
    base = row0 * _C
    pltpu.sync_copy(idx_hbm.at[pl.ds(aligned, _STAGE)], idx_v)
    pltpu.async_copy(table_hbm.at[idx_v.at[off]], rows_v.at[0], gsem)
    pltpu.async_copy(table_hbm.at[idx_v.at[off + 1]], rows_v.at[1], gsem)

    @pl.loop(0, _NCHUNK)
    def _chunk(j):
        buf = lax.rem(j, 4)
        pltpu.make_async_copy(
            table_hbm.at[idx_v.at[off + j]], rows_v.at[buf], gsem
        ).wait()
        pltpu.async_copy(
            rows_v.at[buf], out_hbm.at[pl.ds(base + j * _C, _C)], osem
        )

        @pl.when(j + 2 < _NCHUNK)
        def _start_next():
            nbuf = lax.rem(j + 2, 4)

            @pl.when(j >= 2)
            def _reclaim():
                pltpu.make_async_copy(
                    rows_v.at[nbuf],
                    out_hbm.at[pl.ds(base + (j - 2) * _C, _C)],
                    osem,
                ).wait()

            pltpu.async_copy(
                table_hbm.at[idx_v.at[off + j + 2]], rows_v.at[nbuf], gsem
            )

    for _ in range(4):
        pltpu.make_async_copy(
            rows_v.at[0], out_hbm.at[pl.ds(base, _C)], osem
        ).wait()


def kernel(table, idx):
    idx_flat = idx.reshape(_NROW, _C)
    out = _gather(table, idx_flat)
    return out.reshape(_B, _P, 1, 2, _D)


# speedup vs baseline: 1.6359x; 1.6342x over previous
import functools

import jax
import jax.numpy as jnp
from jax import lax
from jax.experimental import pallas as pl
from jax.experimental.pallas import tpu as pltpu
from jax.experimental.pallas import tpu_sc as plsc

_D = 128
_B = 1024
_P = 50
_PB = _P * 2
_R = _B * _PB
_NC, _NS = 2, 16
_NW = _NC * _NS
_BW = _B // _NW
_NPAIR = _BW // 2

_mesh = plsc.VectorSubcoreMesh(core_axis_name="c", subcore_axis_name="s")


@functools.partial(
    pl.kernel,
    out_type=jax.ShapeDtypeStruct((_R, _D), jnp.float32),
    mesh=_mesh,
    scratch_types=[
        pltpu.VMEM((_BW, _PB), jnp.int32),
        pltpu.VMEM((4, 2 * _PB, _D), jnp.float32),
        pltpu.SemaphoreType.DMA,
        pltpu.SemaphoreType.DMA,
    ],
)
def _gather(table_hbm, idx_hbm, out_hbm, idx_v, rows_v, gsem, osem):
    wid = lax.axis_index("s") * _NC + lax.axis_index("c")
    b0 = wid * _BW
    pltpu.sync_copy(idx_hbm.at[pl.ds(b0, _BW)], idx_v)

    def start_pair(t, buf):
        pltpu.async_copy(
            table_hbm.at[idx_v.at[2 * t]], rows_v.at[buf, pl.ds(0, _PB)], gsem
        )
        pltpu.async_copy(
            table_hbm.at[idx_v.at[2 * t + 1]],
            rows_v.at[buf, pl.ds(_PB, _PB)],
            gsem,
        )

    def wait_pair(t, buf):
        pltpu.make_async_copy(
            table_hbm.at[idx_v.at[2 * t]], rows_v.at[buf, pl.ds(0, _PB)], gsem
        ).wait()
        pltpu.make_async_copy(
            table_hbm.at[idx_v.at[2 * t + 1]],
            rows_v.at[buf, pl.ds(_PB, _PB)],
            gsem,
        ).wait()

    start_pair(0, 0)
    start_pair(1, 1)

    @pl.loop(0, _NPAIR)
    def _pair(t):
        buf = lax.rem(t, 4)
        wait_pair(t, buf)
        pltpu.async_copy(
            rows_v.at[buf], out_hbm.at[pl.ds((b0 + 2 * t) * _PB, 2 * _PB)], osem
        )

        @pl.when(t + 2 < _NPAIR)
        def _start_next():
            nbuf = lax.rem(t + 2, 4)

            @pl.when(t >= 2)
            def _reclaim():
                pltpu.make_async_copy(
                    rows_v.at[nbuf],
                    out_hbm.at[pl.ds((b0 + 2 * (t - 2)) * _PB, 2 * _PB)],
                    osem,
                ).wait()

            start_pair(t + 2, nbuf)

    for _ in range(4):
        pltpu.make_async_copy(
            rows_v.at[0], out_hbm.at[pl.ds(b0 * _PB, 2 * _PB)], osem
        ).wait()


def kernel(table, idx):
    idx_flat = idx.reshape(_B, _PB)
    out = _gather(table, idx_flat)
    return out.reshape(_B, _P, 1, 2, _D)


# speedup vs baseline: 1.6370x; 1.0006x over previous
import functools

import jax
import jax.numpy as jnp
from jax import lax
from jax.experimental import pallas as pl
from jax.experimental.pallas import tpu as pltpu
from jax.experimental.pallas import tpu_sc as plsc

_D = 128
_B = 1024
_P = 50
_PB = _P * 2
_R = _B * _PB
_NC, _NS = 2, 16
_NW = _NC * _NS
_BW = _B // _NW
_NPAIR = _BW // 2

_mesh = plsc.VectorSubcoreMesh(core_axis_name="c", subcore_axis_name="s")


@functools.partial(
    pl.kernel,
    out_type=jax.ShapeDtypeStruct((_R, _D), jnp.float32),
    mesh=_mesh,
    scratch_types=[
        pltpu.VMEM((_BW, _PB), jnp.int32),
        pltpu.VMEM((4, 2 * _PB, _D), jnp.float32),
        pltpu.SemaphoreType.DMA,
        pltpu.SemaphoreType.DMA,
    ],
)
def _gather(table_hbm, idx_hbm, out_hbm, idx_v, rows_v, gsem, osem):
    wid = lax.axis_index("s") * _NC + lax.axis_index("c")
    b0 = wid * _BW
    pltpu.sync_copy(idx_hbm.at[pl.ds(b0, _BW)], idx_v)

    def start_pair(t, buf):
        pltpu.async_copy(
            table_hbm.at[idx_v.at[2 * t]], rows_v.at[buf, pl.ds(0, _PB)], gsem
        )
        pltpu.async_copy(
            table_hbm.at[idx_v.at[2 * t + 1]],
            rows_v.at[buf, pl.ds(_PB, _PB)],
            gsem,
        )

    def wait_pair(t, buf):
        pltpu.make_async_copy(
            table_hbm.at[idx_v.at[2 * t]], rows_v.at[buf, pl.ds(0, _PB)], gsem
        ).wait()
        pltpu.make_async_copy(
            table_hbm.at[idx_v.at[2 * t + 1]],
            rows_v.at[buf, pl.ds(_PB, _PB)],
            gsem,
        ).wait()

    start_pair(0, 0)
    start_pair(1, 1)
    start_pair(2, 2)

    @pl.loop(0, _NPAIR)
    def _pair(t):
        buf = lax.rem(t, 4)
        wait_pair(t, buf)
        pltpu.async_copy(
            rows_v.at[buf], out_hbm.at[pl.ds((b0 + 2 * t) * _PB, 2 * _PB)], osem
        )

        @pl.when(t + 3 < _NPAIR)
        def _start_next():
            nbuf = lax.rem(t + 3, 4)

            @pl.when(t >= 1)
            def _reclaim():
                pltpu.make_async_copy(
                    rows_v.at[nbuf],
                    out_hbm.at[pl.ds((b0 + 2 * (t - 1)) * _PB, 2 * _PB)],
                    osem,
                ).wait()

            start_pair(t + 3, nbuf)

    for _ in range(4):
        pltpu.make_async_copy(
            rows_v.at[0], out_hbm.at[pl.ds(b0 * _PB, 2 * _PB)], osem
        ).wait()


def kernel(table, idx):
    idx_flat = idx.reshape(_B, _PB)
    out = _gather(table, idx_flat)
    return out.reshape(_B, _P, 1, 2, _D)
